# baseline (speedup 1.0000x reference)
import jax
import jax.numpy as jnp
from jax import lax
from jax.experimental import pallas as pl
from jax.experimental.pallas import tpu as pltpu
from jax.experimental.pallas import tpu_sc as plsc

EMBED_DIM = 64
BATCH = 16384

NC = 2
NS = 16
LANES = 16
NW = NC * NS
B_PER_W = BATCH // NW
CHUNK = 128
NCHUNK = B_PER_W // CHUNK


def _body(uidx_hbm, pidx_hbm, nidx_hbm, utab_hbm, itab_hbm,
          pos_hbm, neg_hbm,
          uidx_v, pidx_v, nidx_v, urows, prows, nrows, pos_v, neg_v, sem):
    wid = lax.axis_index("s") * NC + lax.axis_index("c")
    base = wid * B_PER_W

    pltpu.sync_copy(uidx_hbm.at[wid], uidx_v)
    pltpu.sync_copy(pidx_hbm.at[wid], pidx_v)
    pltpu.sync_copy(nidx_hbm.at[wid], nidx_v)

    copies = []
    for c in range(NCHUNK):
        dst = pl.ds(c * CHUNK, CHUNK)
        copies.append(pltpu.async_copy(utab_hbm.at[uidx_v.at[c]], urows.at[dst], sem))
        copies.append(pltpu.async_copy(itab_hbm.at[pidx_v.at[c]], prows.at[dst], sem))
        copies.append(pltpu.async_copy(itab_hbm.at[nidx_v.at[c]], nrows.at[dst], sem))
    for cp in copies:
        cp.wait()

    lane = lax.broadcasted_iota(jnp.int32, (LANES,), 0)
    zero = jnp.zeros((LANES,), jnp.float32)
    NACC = 4

    def gbody(g, carry):
        row = g * LANES + lane
        paccs = [zero] * NACC
        naccs = [zero] * NACC
        for d in range(EMBED_DIM):
            col = (lane + d) & (EMBED_DIM - 1)
            lu = plsc.load_gather(urows, [row, col])
            lp = plsc.load_gather(prows, [row, col])
            ln = plsc.load_gather(nrows, [row, col])
            k = d % NACC
            paccs[k] = paccs[k] + lu * lp
            naccs[k] = naccs[k] + lu * ln
        pacc = (paccs[0] + paccs[1]) + (paccs[2] + paccs[3])
        nacc = (naccs[0] + naccs[1]) + (naccs[2] + naccs[3])
        pos_v[pl.ds(g * LANES, LANES)] = pacc
        neg_v[pl.ds(g * LANES, LANES)] = nacc
        return carry

    lax.fori_loop(0, B_PER_W // LANES, gbody, 0)

    pltpu.sync_copy(pos_v, pos_hbm.at[pl.ds(base, B_PER_W)])
    pltpu.sync_copy(neg_v, neg_hbm.at[pl.ds(base, B_PER_W)])


@jax.jit
def kernel(user_inputs, pos_item_inputs, neg_item_inputs, user_table, item_table):
    mesh = plsc.VectorSubcoreMesh(core_axis_name="c", subcore_axis_name="s")
    uidx = user_inputs.astype(jnp.int32).reshape(NW, NCHUNK, CHUNK)
    pidx = pos_item_inputs.astype(jnp.int32).reshape(NW, NCHUNK, CHUNK)
    nidx = neg_item_inputs.astype(jnp.int32).reshape(NW, NCHUNK, CHUNK)
    run = pl.kernel(
        _body,
        out_type=(jax.ShapeDtypeStruct((BATCH,), jnp.float32),
                  jax.ShapeDtypeStruct((BATCH,), jnp.float32)),
        mesh=mesh,
        compiler_params=pltpu.CompilerParams(
            use_tc_tiling_on_sc=False, needs_layout_passes=False),
        scratch_types=[
            pltpu.VMEM((NCHUNK, CHUNK), jnp.int32),
            pltpu.VMEM((NCHUNK, CHUNK), jnp.int32),
            pltpu.VMEM((NCHUNK, CHUNK), jnp.int32),
            pltpu.VMEM((B_PER_W, EMBED_DIM), jnp.float32),
            pltpu.VMEM((B_PER_W, EMBED_DIM), jnp.float32),
            pltpu.VMEM((B_PER_W, EMBED_DIM), jnp.float32),
            pltpu.VMEM((B_PER_W,), jnp.float32),
            pltpu.VMEM((B_PER_W,), jnp.float32),
            pltpu.SemaphoreType.DMA,
        ],
    )
    return run(uidx, pidx, nidx, user_table, item_table)


# speedup vs baseline: 1.0622x; 1.0622x over previous
import jax
import jax.numpy as jnp
from jax import lax
from jax.experimental import pallas as pl
from jax.experimental.pallas import tpu as pltpu
from jax.experimental.pallas import tpu_sc as plsc

EMBED_DIM = 64
BATCH = 16384

NC = 2
NS = 16
LANES = 16
NW = NC * NS
B_PER_W = BATCH // NW
CHUNK = 128
NCHUNK = B_PER_W // CHUNK
NBUF = 2
PAIR_W = 2 * EMBED_DIM


def _body(upair_hbm, ppair_hbm, npair_hbm, uhalf_hbm, phalf_hbm, nhalf_hbm,
          utab_hbm, itab_hbm,
          pos_hbm, neg_hbm,
          uiv, piv, niv, uhv, phv, nhv, ub, pb, nb, pos_v, neg_v,
          isem, sem0, sem1):
    wid = lax.axis_index("s") * NC + lax.axis_index("c")
    base = wid * B_PER_W
    k0 = wid * NCHUNK
    sems = (sem0, sem1)

    icopies = []
    for s in range(NCHUNK):
        icopies.append(pltpu.async_copy(upair_hbm.at[k0 + s], uiv.at[s], isem))
        icopies.append(pltpu.async_copy(ppair_hbm.at[k0 + s], piv.at[s], isem))
        icopies.append(pltpu.async_copy(npair_hbm.at[k0 + s], niv.at[s], isem))
        icopies.append(pltpu.async_copy(uhalf_hbm.at[k0 + s], uhv.at[s], isem))
        icopies.append(pltpu.async_copy(phalf_hbm.at[k0 + s], phv.at[s], isem))
        icopies.append(pltpu.async_copy(nhalf_hbm.at[k0 + s], nhv.at[s], isem))
    for cp in icopies:
        cp.wait()

    def fire(s):
        j = s % NBUF
        sem = sems[j]
        return (pltpu.async_copy(utab_hbm.at[uiv.at[s]], ub.at[j], sem),
                pltpu.async_copy(itab_hbm.at[piv.at[s]], pb.at[j], sem),
                pltpu.async_copy(itab_hbm.at[niv.at[s]], nb.at[j], sem))

    lane = lax.broadcasted_iota(jnp.int32, (LANES,), 0)
    zero = jnp.zeros((LANES,), jnp.float32)
    NACC = 4

    inflight = fire(0)
    for s in range(NCHUNK):
        for cp in inflight:
            cp.wait()
        if s + 1 < NCHUNK:
            inflight = fire(s + 1)
        j = s % NBUF
        jvec = jnp.full((LANES,), j, jnp.int32)

        def gbody(g, carry, _j=jvec, _s=s):
            row = g * LANES + lane
            hu = uhv[_s, pl.ds(g * LANES, LANES)]
            hp = phv[_s, pl.ds(g * LANES, LANES)]
            hn = nhv[_s, pl.ds(g * LANES, LANES)]
            paccs = [zero] * NACC
            naccs = [zero] * NACC
            for d in range(EMBED_DIM):
                col = (lane + d) & (EMBED_DIM - 1)
                lu = plsc.load_gather(ub, [_j, row, col + hu])
                lp = plsc.load_gather(pb, [_j, row, col + hp])
                ln = plsc.load_gather(nb, [_j, row, col + hn])
                k = d % NACC
                paccs[k] = paccs[k] + lu * lp
                naccs[k] = naccs[k] + lu * ln
            pacc = (paccs[0] + paccs[1]) + (paccs[2] + paccs[3])
            nacc = (naccs[0] + naccs[1]) + (naccs[2] + naccs[3])
            off = _s * CHUNK + g * LANES
            pos_v[pl.ds(off, LANES)] = pacc
            neg_v[pl.ds(off, LANES)] = nacc
            return carry

        lax.fori_loop(0, CHUNK // LANES, gbody, 0)

    pltpu.sync_copy(pos_v, pos_hbm.at[pl.ds(base, B_PER_W)])
    pltpu.sync_copy(neg_v, neg_hbm.at[pl.ds(base, B_PER_W)])


@jax.jit
def kernel(user_inputs, pos_item_inputs, neg_item_inputs, user_table, item_table):
    mesh = plsc.VectorSubcoreMesh(core_axis_name="c", subcore_axis_name="s")
    nrow = BATCH // CHUNK

    def prep(idx):
        idx = idx.astype(jnp.int32)
        return ((idx >> 1).reshape(nrow, CHUNK),
                ((idx & 1) << 6).reshape(nrow, CHUNK))

    upair, uhalf = prep(user_inputs)
    ppair, phalf = prep(pos_item_inputs)
    npair, nhalf = prep(neg_item_inputs)
    utab = user_table.reshape(-1, PAIR_W)
    itab = item_table.reshape(-1, PAIR_W)
    run = pl.kernel(
        _body,
        out_type=(jax.ShapeDtypeStruct((BATCH,), jnp.float32),
                  jax.ShapeDtypeStruct((BATCH,), jnp.float32)),
        mesh=mesh,
        compiler_params=pltpu.CompilerParams(needs_layout_passes=False),
        scratch_types=[
            pltpu.VMEM((NCHUNK, CHUNK), jnp.int32),
            pltpu.VMEM((NCHUNK, CHUNK), jnp.int32),
            pltpu.VMEM((NCHUNK, CHUNK), jnp.int32),
            pltpu.VMEM((NCHUNK, CHUNK), jnp.int32),
            pltpu.VMEM((NCHUNK, CHUNK), jnp.int32),
            pltpu.VMEM((NCHUNK, CHUNK), jnp.int32),
            pltpu.VMEM((NBUF, CHUNK, PAIR_W), jnp.float32),
            pltpu.VMEM((NBUF, CHUNK, PAIR_W), jnp.float32),
            pltpu.VMEM((NBUF, CHUNK, PAIR_W), jnp.float32),
            pltpu.VMEM((B_PER_W,), jnp.float32),
            pltpu.VMEM((B_PER_W,), jnp.float32),
            pltpu.SemaphoreType.DMA,
            pltpu.SemaphoreType.DMA,
            pltpu.SemaphoreType.DMA,
        ],
    )
    return run(upair, ppair, npair, uhalf, phalf, nhalf, utab, itab)
